# baseline (speedup 1.0000x reference)
import functools

import jax
import jax.numpy as jnp
from jax import lax
from jax.experimental import pallas as pl
from jax.experimental.pallas import tpu as pltpu
from jax.experimental.pallas import tpu_sc as plsc

NUM_USERS = 1000000
NUM_PRODUCTS = 1000000
EMBED = 64
BATCH = 16384

NC, NS, LANES = 2, 16, 16
NW = NC * NS
ROWS_PER_W = BATCH // NW
GCHUNK = 128
NCHUNK = ROWS_PER_W // GCHUNK

BLK = 16384
SC_CHUNK = 384
SC_CHUNKS_PER_W = 16
CSC = NW * SC_CHUNKS_PER_W * SC_CHUNK
TC_BLK0 = CSC // BLK


def _tc_matvec(ut_ref, pt_ref, wu_ref, wp_ref, vu_ref, vp_ref):
    vu_ref[...] = jnp.sum(ut_ref[...] * wu_ref[...], axis=0)
    vp_ref[...] = jnp.sum(pt_ref[...] * wp_ref[...], axis=0)


def _sc_matvec(ut, pt, wu_h, wp_h, vu_sc, vp_sc,
               vmA_u, vmA_p, vmB_u, vmB_p, o_u, o_p, wu_v, wp_v,
               semA, semB):
    wid = lax.axis_index("s") * NC + lax.axis_index("c")
    cbase = wid * (SC_CHUNKS_PER_W * SC_CHUNK)
    last = SC_CHUNKS_PER_W - 1

    pltpu.sync_copy(wu_h, wu_v)
    pltpu.sync_copy(wp_h, wp_v)
    wu_c = [wu_v[pl.ds(c * LANES, LANES)] for c in range(EMBED // LANES)]
    wp_c = [wp_v[pl.ds(c * LANES, LANES)] for c in range(EMBED // LANES)]
    ws_u = [wu_c[e // LANES][e % LANES] for e in range(EMBED)]
    ws_p = [wp_c[e // LANES][e % LANES] for e in range(EMBED)]

    def issue(c, bu, bp, sem):
        c0 = cbase + c * SC_CHUNK
        pltpu.async_copy(ut.at[:, pl.ds(c0, SC_CHUNK)], bu, sem)
        pltpu.async_copy(pt.at[:, pl.ds(c0, SC_CHUNK)], bp, sem)

    def drain(bu, bp, sem):
        pltpu.make_async_copy(ut.at[:, pl.ds(0, SC_CHUNK)], bu, sem).wait()
        pltpu.make_async_copy(pt.at[:, pl.ds(0, SC_CHUNK)], bp, sem).wait()

    def compute(c, bu, bp):
        def group_body(g, carry2):
            sl = pl.ds(g * LANES, LANES)
            au = bu[0, sl] * ws_u[0]
            ap = bp[0, sl] * ws_p[0]
            for e in range(1, EMBED):
                au = au + bu[e, sl] * ws_u[e]
                ap = ap + bp[e, sl] * ws_p[e]
            o_u[sl] = au
            o_p[sl] = ap
            return carry2

        lax.fori_loop(0, SC_CHUNK // LANES, group_body, 0)
        c0 = cbase + c * SC_CHUNK
        pltpu.sync_copy(o_u, vu_sc.at[pl.ds(c0, SC_CHUNK)])
        pltpu.sync_copy(o_p, vp_sc.at[pl.ds(c0, SC_CHUNK)])

    issue(0, vmA_u, vmA_p, semA)
    issue(1, vmB_u, vmB_p, semB)

    def pair_body(i, carry):
        cA = 2 * i
        drain(vmA_u, vmA_p, semA)
        compute(cA, vmA_u, vmA_p)
        issue(jnp.minimum(cA + 2, last), vmA_u, vmA_p, semA)
        cB = 2 * i + 1
        drain(vmB_u, vmB_p, semB)
        compute(cB, vmB_u, vmB_p)
        issue(jnp.minimum(cB + 2, last), vmB_u, vmB_p, semB)
        return carry

    lax.fori_loop(0, SC_CHUNKS_PER_W // 2, pair_body, 0)
    drain(vmA_u, vmA_p, semA)
    drain(vmB_u, vmB_p, semB)


def _sc_gather(uid2, pid2, vu_sc_h, vp_sc_h, vu_tc_h, vp_tc_h, bvec,
               out_hbm, idx_u, idx_p, g_lo, g_hi, out_v, b_v, sem):
    wid = lax.axis_index("s") * NC + lax.axis_index("c")

    pltpu.sync_copy(uid2.at[pl.ds(wid * NCHUNK, NCHUNK)],
                    idx_u.at[pl.ds(0, NCHUNK)])
    pltpu.sync_copy(pid2.at[pl.ds(wid * NCHUNK, NCHUNK)],
                    idx_p.at[pl.ds(0, NCHUNK)])
    pltpu.sync_copy(bvec, b_v)

    def clamp_body(j, carry):
        k = j // (GCHUNK // LANES)
        col = (j % (GCHUNK // LANES)) * LANES
        sl = pl.ds(col, LANES)
        for ref in (idx_u, idx_p):
            iv = ref[k, sl]
            in_sc = iv < CSC
            ref[NCHUNK + k, sl] = jnp.where(in_sc, iv, 0)
            ref[2 * NCHUNK + k, sl] = jnp.where(in_sc, CSC, iv)
        return carry

    lax.fori_loop(0, NCHUNK * (GCHUNK // LANES), clamp_body, 0)

    copies = []
    for k in range(NCHUNK):
        copies.append(pltpu.async_copy(
            vu_sc_h.at[idx_u.at[NCHUNK + k]],
            g_lo.at[pl.ds(k * GCHUNK, GCHUNK)], sem))
        copies.append(pltpu.async_copy(
            vp_sc_h.at[idx_p.at[NCHUNK + k]],
            g_lo.at[pl.ds((NCHUNK + k) * GCHUNK, GCHUNK)], sem))
        copies.append(pltpu.async_copy(
            vu_tc_h.at[idx_u.at[2 * NCHUNK + k]],
            g_hi.at[pl.ds(k * GCHUNK, GCHUNK)], sem))
        copies.append(pltpu.async_copy(
            vp_tc_h.at[idx_p.at[2 * NCHUNK + k]],
            g_hi.at[pl.ds((NCHUNK + k) * GCHUNK, GCHUNK)], sem))
    for c in copies:
        c.wait()

    b_l = b_v[...]

    def block_body(g, carry):
        sl = pl.ds(g * LANES, LANES)
        slp = pl.ds(ROWS_PER_W + g * LANES, LANES)
        row = (g * LANES) // GCHUNK
        col = (g * LANES) % GCHUNK
        ku = idx_u[row, pl.ds(col, LANES)] < CSC
        kp = idx_p[row, pl.ds(col, LANES)] < CSC
        gu = jnp.where(ku, g_lo[sl], g_hi[sl])
        gp = jnp.where(kp, g_lo[slp], g_hi[slp])
        out_v[sl] = 1.0 / (1.0 + jnp.exp(-(gu + gp + b_l)))
        return carry

    lax.fori_loop(0, ROWS_PER_W // LANES, block_body, 0)
    pltpu.sync_copy(out_v, out_hbm.at[pl.ds(wid * ROWS_PER_W, ROWS_PER_W)])


def kernel(user_ids, product_ids, user_table, product_table, W, b):
    wu = W[:EMBED, :]
    wp = W[EMBED:, :]
    wu1 = W[:EMBED, 0]
    wp1 = W[EMBED:, 0]
    bvec = jnp.broadcast_to(b, (LANES,)).astype(jnp.float32)
    ut = user_table.T
    pt = product_table.T

    sc_mesh = plsc.VectorSubcoreMesh(core_axis_name="c", subcore_axis_name="s")
    sc_mv = functools.partial(
        pl.kernel, mesh=sc_mesh,
        compiler_params=pltpu.CompilerParams(
            needs_layout_passes=False, use_tc_tiling_on_sc=True),
        out_type=[
            jax.ShapeDtypeStruct((CSC,), jnp.float32),
            jax.ShapeDtypeStruct((CSC,), jnp.float32),
        ],
        scratch_types=[
            pltpu.VMEM((EMBED, SC_CHUNK), jnp.float32),
            pltpu.VMEM((EMBED, SC_CHUNK), jnp.float32),
            pltpu.VMEM((EMBED, SC_CHUNK), jnp.float32),
            pltpu.VMEM((EMBED, SC_CHUNK), jnp.float32),
            pltpu.VMEM((SC_CHUNK,), jnp.float32),
            pltpu.VMEM((SC_CHUNK,), jnp.float32),
            pltpu.VMEM((EMBED,), jnp.float32),
            pltpu.VMEM((EMBED,), jnp.float32),
            pltpu.SemaphoreType.DMA,
            pltpu.SemaphoreType.DMA,
        ],
    )(_sc_matvec)
    vu_sc, vp_sc = sc_mv(ut, pt, wu1, wp1)

    nblk_tc = (NUM_USERS - CSC + BLK - 1) // BLK
    vu_tc, vp_tc = pl.pallas_call(
        _tc_matvec,
        grid=(nblk_tc,),
        in_specs=[
            pl.BlockSpec((EMBED, BLK), lambda i: (0, i + TC_BLK0)),
            pl.BlockSpec((EMBED, BLK), lambda i: (0, i + TC_BLK0)),
            pl.BlockSpec((EMBED, 1), lambda i: (0, 0)),
            pl.BlockSpec((EMBED, 1), lambda i: (0, 0)),
        ],
        out_specs=[
            pl.BlockSpec((BLK,), lambda i: (i + TC_BLK0,)),
            pl.BlockSpec((BLK,), lambda i: (i + TC_BLK0,)),
        ],
        out_shape=[
            jax.ShapeDtypeStruct((NUM_USERS,), jnp.float32),
            jax.ShapeDtypeStruct((NUM_PRODUCTS,), jnp.float32),
        ],
    )(ut, pt, wu, wp)

    uid2 = user_ids.reshape(BATCH // GCHUNK, GCHUNK)
    pid2 = product_ids.reshape(BATCH // GCHUNK, GCHUNK)

    run = functools.partial(
        pl.kernel, mesh=sc_mesh,
        compiler_params=pltpu.CompilerParams(
            needs_layout_passes=False, use_tc_tiling_on_sc=False),
        out_type=jax.ShapeDtypeStruct((BATCH,), jnp.float32),
        scratch_types=[
            pltpu.VMEM((3 * NCHUNK, GCHUNK), jnp.int32),
            pltpu.VMEM((3 * NCHUNK, GCHUNK), jnp.int32),
            pltpu.VMEM((2 * ROWS_PER_W,), jnp.float32),
            pltpu.VMEM((2 * ROWS_PER_W,), jnp.float32),
            pltpu.VMEM((ROWS_PER_W,), jnp.float32),
            pltpu.VMEM((LANES,), jnp.float32),
            pltpu.SemaphoreType.DMA,
        ],
    )(_sc_gather)
    out = run(uid2, pid2, vu_sc, vp_sc, vu_tc, vp_tc, bvec)
    return out.reshape(BATCH, 1)


# speedup vs baseline: 1.2983x; 1.2983x over previous
import functools

import jax
import jax.numpy as jnp
from jax import lax
from jax.experimental import pallas as pl
from jax.experimental.pallas import tpu as pltpu
from jax.experimental.pallas import tpu_sc as plsc

NUM_USERS = 1000000
NUM_PRODUCTS = 1000000
EMBED = 64
BATCH = 16384

NC, NS, LANES = 2, 16, 16
NW = NC * NS
ROWS_PER_W = BATCH // NW
GCHUNK = 128
NCHUNK = ROWS_PER_W // GCHUNK

BLK = 16384
SC_CHUNK = 384
SC_CHUNKS_PER_W = 24
CSC = NW * SC_CHUNKS_PER_W * SC_CHUNK
TC_BLK0 = CSC // BLK


def _tc_matvec(ut_ref, pt_ref, wu_ref, wp_ref, vu_ref, vp_ref):
    vu_ref[...] = jnp.sum(ut_ref[...] * wu_ref[...], axis=0)
    vp_ref[...] = jnp.sum(pt_ref[...] * wp_ref[...], axis=0)


def _sc_matvec(ut, pt, wu_h, wp_h, vu_sc, vp_sc,
               vmA_u, vmA_p, vmB_u, vmB_p, o_u, o_p, wu_v, wp_v,
               semA, semB):
    wid = lax.axis_index("s") * NC + lax.axis_index("c")
    cbase = wid * (SC_CHUNKS_PER_W * SC_CHUNK)
    last = SC_CHUNKS_PER_W - 1

    pltpu.sync_copy(wu_h, wu_v)
    pltpu.sync_copy(wp_h, wp_v)
    wu_c = [wu_v[pl.ds(c * LANES, LANES)] for c in range(EMBED // LANES)]
    wp_c = [wp_v[pl.ds(c * LANES, LANES)] for c in range(EMBED // LANES)]
    ws_u = [wu_c[e // LANES][e % LANES] for e in range(EMBED)]
    ws_p = [wp_c[e // LANES][e % LANES] for e in range(EMBED)]

    def issue(c, bu, bp, sem):
        c0 = cbase + c * SC_CHUNK
        pltpu.async_copy(ut.at[:, pl.ds(c0, SC_CHUNK)], bu, sem)
        pltpu.async_copy(pt.at[:, pl.ds(c0, SC_CHUNK)], bp, sem)

    def drain(bu, bp, sem):
        pltpu.make_async_copy(ut.at[:, pl.ds(0, SC_CHUNK)], bu, sem).wait()
        pltpu.make_async_copy(pt.at[:, pl.ds(0, SC_CHUNK)], bp, sem).wait()

    def compute(c, bu, bp):
        def group_body(g, carry2):
            sl = pl.ds(g * LANES, LANES)
            au = bu[0, sl] * ws_u[0]
            ap = bp[0, sl] * ws_p[0]
            for e in range(1, EMBED):
                au = au + bu[e, sl] * ws_u[e]
                ap = ap + bp[e, sl] * ws_p[e]
            o_u[sl] = au
            o_p[sl] = ap
            return carry2

        lax.fori_loop(0, SC_CHUNK // LANES, group_body, 0)
        c0 = cbase + c * SC_CHUNK
        pltpu.sync_copy(o_u, vu_sc.at[pl.ds(c0, SC_CHUNK)])
        pltpu.sync_copy(o_p, vp_sc.at[pl.ds(c0, SC_CHUNK)])

    issue(0, vmA_u, vmA_p, semA)
    issue(1, vmB_u, vmB_p, semB)

    def pair_body(i, carry):
        cA = 2 * i
        drain(vmA_u, vmA_p, semA)
        compute(cA, vmA_u, vmA_p)
        issue(jnp.minimum(cA + 2, last), vmA_u, vmA_p, semA)
        cB = 2 * i + 1
        drain(vmB_u, vmB_p, semB)
        compute(cB, vmB_u, vmB_p)
        issue(jnp.minimum(cB + 2, last), vmB_u, vmB_p, semB)
        return carry

    lax.fori_loop(0, SC_CHUNKS_PER_W // 2, pair_body, 0)
    drain(vmA_u, vmA_p, semA)
    drain(vmB_u, vmB_p, semB)


def _sc_gather(uid2, pid2, vu_h, vp_h, bvec,
               out_hbm, idx_u, idx_p, gbuf, out_v, b_v, sem):
    wid = lax.axis_index("s") * NC + lax.axis_index("c")

    pltpu.sync_copy(uid2.at[pl.ds(wid * NCHUNK, NCHUNK)], idx_u)
    pltpu.sync_copy(pid2.at[pl.ds(wid * NCHUNK, NCHUNK)], idx_p)
    pltpu.sync_copy(bvec, b_v)

    copies = []
    for k in range(NCHUNK):
        copies.append(pltpu.async_copy(
            vu_h.at[idx_u.at[k]],
            gbuf.at[pl.ds(k * GCHUNK, GCHUNK)], sem))
        copies.append(pltpu.async_copy(
            vp_h.at[idx_p.at[k]],
            gbuf.at[pl.ds((NCHUNK + k) * GCHUNK, GCHUNK)], sem))
    for c in copies:
        c.wait()

    b_l = b_v[...]

    def block_body(g, carry):
        sl = pl.ds(g * LANES, LANES)
        slp = pl.ds(ROWS_PER_W + g * LANES, LANES)
        out_v[sl] = 1.0 / (1.0 + jnp.exp(-(gbuf[sl] + gbuf[slp] + b_l)))
        return carry

    lax.fori_loop(0, ROWS_PER_W // LANES, block_body, 0)
    pltpu.sync_copy(out_v, out_hbm.at[pl.ds(wid * ROWS_PER_W, ROWS_PER_W)])


def kernel(user_ids, product_ids, user_table, product_table, W, b):
    wu = W[:EMBED, :]
    wp = W[EMBED:, :]
    wu1 = W[:EMBED, 0]
    wp1 = W[EMBED:, 0]
    bvec = jnp.broadcast_to(b, (LANES,)).astype(jnp.float32)
    ut = user_table.T
    pt = product_table.T

    sc_mesh = plsc.VectorSubcoreMesh(core_axis_name="c", subcore_axis_name="s")
    sc_mv = functools.partial(
        pl.kernel, mesh=sc_mesh,
        compiler_params=pltpu.CompilerParams(
            needs_layout_passes=False, use_tc_tiling_on_sc=True),
        out_type=[
            jax.ShapeDtypeStruct((CSC,), jnp.float32),
            jax.ShapeDtypeStruct((CSC,), jnp.float32),
        ],
        scratch_types=[
            pltpu.VMEM((EMBED, SC_CHUNK), jnp.float32),
            pltpu.VMEM((EMBED, SC_CHUNK), jnp.float32),
            pltpu.VMEM((EMBED, SC_CHUNK), jnp.float32),
            pltpu.VMEM((EMBED, SC_CHUNK), jnp.float32),
            pltpu.VMEM((SC_CHUNK,), jnp.float32),
            pltpu.VMEM((SC_CHUNK,), jnp.float32),
            pltpu.VMEM((EMBED,), jnp.float32),
            pltpu.VMEM((EMBED,), jnp.float32),
            pltpu.SemaphoreType.DMA,
            pltpu.SemaphoreType.DMA,
        ],
    )(_sc_matvec)
    vu_sc, vp_sc = sc_mv(ut, pt, wu1, wp1)

    nblk_tc = (NUM_USERS - CSC + BLK - 1) // BLK
    vu_tc, vp_tc = pl.pallas_call(
        _tc_matvec,
        grid=(nblk_tc,),
        in_specs=[
            pl.BlockSpec((EMBED, BLK), lambda i: (0, i + TC_BLK0)),
            pl.BlockSpec((EMBED, BLK), lambda i: (0, i + TC_BLK0)),
            pl.BlockSpec((EMBED, 1), lambda i: (0, 0)),
            pl.BlockSpec((EMBED, 1), lambda i: (0, 0)),
        ],
        out_specs=[
            pl.BlockSpec((BLK,), lambda i: (i + TC_BLK0,)),
            pl.BlockSpec((BLK,), lambda i: (i + TC_BLK0,)),
        ],
        out_shape=[
            jax.ShapeDtypeStruct((NUM_USERS,), jnp.float32),
            jax.ShapeDtypeStruct((NUM_PRODUCTS,), jnp.float32),
        ],
    )(ut, pt, wu, wp)

    vu = lax.dynamic_update_slice(vu_tc, vu_sc, (0,))
    vp = lax.dynamic_update_slice(vp_tc, vp_sc, (0,))

    uid2 = user_ids.reshape(BATCH // GCHUNK, GCHUNK)
    pid2 = product_ids.reshape(BATCH // GCHUNK, GCHUNK)

    run = functools.partial(
        pl.kernel, mesh=sc_mesh,
        compiler_params=pltpu.CompilerParams(
            needs_layout_passes=False, use_tc_tiling_on_sc=False),
        out_type=jax.ShapeDtypeStruct((BATCH,), jnp.float32),
        scratch_types=[
            pltpu.VMEM((NCHUNK, GCHUNK), jnp.int32),
            pltpu.VMEM((NCHUNK, GCHUNK), jnp.int32),
            pltpu.VMEM((2 * ROWS_PER_W,), jnp.float32),
            pltpu.VMEM((ROWS_PER_W,), jnp.float32),
            pltpu.VMEM((LANES,), jnp.float32),
            pltpu.SemaphoreType.DMA,
        ],
    )(_sc_gather)
    out = run(uid2, pid2, vu, vp, bvec)
    return out.reshape(BATCH, 1)


# speedup vs baseline: 1.3031x; 1.0037x over previous
import functools

import jax
import jax.numpy as jnp
from jax import lax
from jax.experimental import pallas as pl
from jax.experimental.pallas import tpu as pltpu
from jax.experimental.pallas import tpu_sc as plsc

NUM_USERS = 1000000
NUM_PRODUCTS = 1000000
EMBED = 64
BATCH = 16384

NC, NS, LANES = 2, 16, 16
NW = NC * NS
ROWS_PER_W = BATCH // NW
GCHUNK = 128
NCHUNK = ROWS_PER_W // GCHUNK

BLK = 16384
SC_CHUNK = 384
SC_CHUNKS_PER_W = 28
CSC = NW * SC_CHUNKS_PER_W * SC_CHUNK
TC_BLK0 = CSC // BLK


def _tc_matvec(ut_ref, pt_ref, wu_ref, wp_ref, vu_ref, vp_ref):
    vu_ref[...] = jnp.sum(ut_ref[...] * wu_ref[...], axis=0)
    vp_ref[...] = jnp.sum(pt_ref[...] * wp_ref[...], axis=0)


def _sc_matvec(ut, pt, wu_h, wp_h, vu_sc, vp_sc,
               vmA_u, vmA_p, vmB_u, vmB_p, o_u, o_p, wu_v, wp_v,
               semA, semB):
    wid = lax.axis_index("s") * NC + lax.axis_index("c")
    cbase = wid * (SC_CHUNKS_PER_W * SC_CHUNK)
    last = SC_CHUNKS_PER_W - 1

    pltpu.sync_copy(wu_h, wu_v)
    pltpu.sync_copy(wp_h, wp_v)
    wu_c = [wu_v[pl.ds(c * LANES, LANES)] for c in range(EMBED // LANES)]
    wp_c = [wp_v[pl.ds(c * LANES, LANES)] for c in range(EMBED // LANES)]
    ws_u = [wu_c[e // LANES][e % LANES] for e in range(EMBED)]
    ws_p = [wp_c[e // LANES][e % LANES] for e in range(EMBED)]

    def issue(c, bu, bp, sem):
        c0 = cbase + c * SC_CHUNK
        pltpu.async_copy(ut.at[:, pl.ds(c0, SC_CHUNK)], bu, sem)
        pltpu.async_copy(pt.at[:, pl.ds(c0, SC_CHUNK)], bp, sem)

    def drain(bu, bp, sem):
        pltpu.make_async_copy(ut.at[:, pl.ds(0, SC_CHUNK)], bu, sem).wait()
        pltpu.make_async_copy(pt.at[:, pl.ds(0, SC_CHUNK)], bp, sem).wait()

    def compute(c, bu, bp):
        def group_body(g, carry2):
            sl = pl.ds(g * LANES, LANES)
            au = bu[0, sl] * ws_u[0]
            ap = bp[0, sl] * ws_p[0]
            for e in range(1, EMBED):
                au = au + bu[e, sl] * ws_u[e]
                ap = ap + bp[e, sl] * ws_p[e]
            o_u[sl] = au
            o_p[sl] = ap
            return carry2

        lax.fori_loop(0, SC_CHUNK // LANES, group_body, 0)
        c0 = cbase + c * SC_CHUNK
        pltpu.sync_copy(o_u, vu_sc.at[pl.ds(c0, SC_CHUNK)])
        pltpu.sync_copy(o_p, vp_sc.at[pl.ds(c0, SC_CHUNK)])

    issue(0, vmA_u, vmA_p, semA)
    issue(1, vmB_u, vmB_p, semB)

    def pair_body(i, carry):
        cA = 2 * i
        drain(vmA_u, vmA_p, semA)
        compute(cA, vmA_u, vmA_p)
        issue(jnp.minimum(cA + 2, last), vmA_u, vmA_p, semA)
        cB = 2 * i + 1
        drain(vmB_u, vmB_p, semB)
        compute(cB, vmB_u, vmB_p)
        issue(jnp.minimum(cB + 2, last), vmB_u, vmB_p, semB)
        return carry

    lax.fori_loop(0, SC_CHUNKS_PER_W // 2, pair_body, 0)
    drain(vmA_u, vmA_p, semA)
    drain(vmB_u, vmB_p, semB)


def _sc_gather(uid2, pid2, vu_h, vp_h, bvec,
               out_hbm, idx_u, idx_p, gbuf, out_v, b_v, sem):
    wid = lax.axis_index("s") * NC + lax.axis_index("c")

    pltpu.sync_copy(uid2.at[pl.ds(wid * NCHUNK, NCHUNK)], idx_u)
    pltpu.sync_copy(pid2.at[pl.ds(wid * NCHUNK, NCHUNK)], idx_p)
    pltpu.sync_copy(bvec, b_v)

    copies = []
    for k in range(NCHUNK):
        copies.append(pltpu.async_copy(
            vu_h.at[idx_u.at[k]],
            gbuf.at[pl.ds(k * GCHUNK, GCHUNK)], sem))
        copies.append(pltpu.async_copy(
            vp_h.at[idx_p.at[k]],
            gbuf.at[pl.ds((NCHUNK + k) * GCHUNK, GCHUNK)], sem))
    for c in copies:
        c.wait()

    b_l = b_v[...]

    def block_body(g, carry):
        sl = pl.ds(g * LANES, LANES)
        slp = pl.ds(ROWS_PER_W + g * LANES, LANES)
        out_v[sl] = 1.0 / (1.0 + jnp.exp(-(gbuf[sl] + gbuf[slp] + b_l)))
        return carry

    lax.fori_loop(0, ROWS_PER_W // LANES, block_body, 0)
    pltpu.sync_copy(out_v, out_hbm.at[pl.ds(wid * ROWS_PER_W, ROWS_PER_W)])


def kernel(user_ids, product_ids, user_table, product_table, W, b):
    wu = W[:EMBED, :]
    wp = W[EMBED:, :]
    wu1 = W[:EMBED, 0]
    wp1 = W[EMBED:, 0]
    bvec = jnp.broadcast_to(b, (LANES,)).astype(jnp.float32)
    ut = user_table.T
    pt = product_table.T

    sc_mesh = plsc.VectorSubcoreMesh(core_axis_name="c", subcore_axis_name="s")
    sc_mv = functools.partial(
        pl.kernel, mesh=sc_mesh,
        compiler_params=pltpu.CompilerParams(
            needs_layout_passes=False, use_tc_tiling_on_sc=True),
        out_type=[
            jax.ShapeDtypeStruct((CSC,), jnp.float32),
            jax.ShapeDtypeStruct((CSC,), jnp.float32),
        ],
        scratch_types=[
            pltpu.VMEM((EMBED, SC_CHUNK), jnp.float32),
            pltpu.VMEM((EMBED, SC_CHUNK), jnp.float32),
            pltpu.VMEM((EMBED, SC_CHUNK), jnp.float32),
            pltpu.VMEM((EMBED, SC_CHUNK), jnp.float32),
            pltpu.VMEM((SC_CHUNK,), jnp.float32),
            pltpu.VMEM((SC_CHUNK,), jnp.float32),
            pltpu.VMEM((EMBED,), jnp.float32),
            pltpu.VMEM((EMBED,), jnp.float32),
            pltpu.SemaphoreType.DMA,
            pltpu.SemaphoreType.DMA,
        ],
    )(_sc_matvec)
    vu_sc, vp_sc = sc_mv(ut, pt, wu1, wp1)

    nblk_tc = (NUM_USERS - CSC + BLK - 1) // BLK
    vu_tc, vp_tc = pl.pallas_call(
        _tc_matvec,
        grid=(nblk_tc,),
        in_specs=[
            pl.BlockSpec((EMBED, BLK), lambda i: (0, i + TC_BLK0)),
            pl.BlockSpec((EMBED, BLK), lambda i: (0, i + TC_BLK0)),
            pl.BlockSpec((EMBED, 1), lambda i: (0, 0)),
            pl.BlockSpec((EMBED, 1), lambda i: (0, 0)),
        ],
        out_specs=[
            pl.BlockSpec((BLK,), lambda i: (i + TC_BLK0,)),
            pl.BlockSpec((BLK,), lambda i: (i + TC_BLK0,)),
        ],
        out_shape=[
            jax.ShapeDtypeStruct((NUM_USERS,), jnp.float32),
            jax.ShapeDtypeStruct((NUM_PRODUCTS,), jnp.float32),
        ],
    )(ut, pt, wu, wp)

    vu = lax.dynamic_update_slice(vu_tc, vu_sc, (0,))
    vp = lax.dynamic_update_slice(vp_tc, vp_sc, (0,))

    uid2 = user_ids.reshape(BATCH // GCHUNK, GCHUNK)
    pid2 = product_ids.reshape(BATCH // GCHUNK, GCHUNK)

    run = functools.partial(
        pl.kernel, mesh=sc_mesh,
        compiler_params=pltpu.CompilerParams(
            needs_layout_passes=False, use_tc_tiling_on_sc=False),
        out_type=jax.ShapeDtypeStruct((BATCH,), jnp.float32),
        scratch_types=[
            pltpu.VMEM((NCHUNK, GCHUNK), jnp.int32),
            pltpu.VMEM((NCHUNK, GCHUNK), jnp.int32),
            pltpu.VMEM((2 * ROWS_PER_W,), jnp.float32),
            pltpu.VMEM((ROWS_PER_W,), jnp.float32),
            pltpu.VMEM((LANES,), jnp.float32),
            pltpu.SemaphoreType.DMA,
        ],
    )(_sc_gather)
    out = run(uid2, pid2, vu, vp, bvec)
    return out.reshape(BATCH, 1)
